# baseline (speedup 1.0000x reference)
import functools

import jax
import jax.numpy as jnp
from jax import lax
from jax.experimental import pallas as pl
from jax.experimental.pallas import tpu as pltpu
from jax.experimental.pallas import tpu_sc as plsc

_B, _C, _HW = 64, 768, 576
_BSC = 32
_BBLK = 4
_RCHUNK = 64
_NCHUNK = _HW // _RCHUNK
_NV = _C // 16


def _sc_pool_body(x_hbm, out_hbm, buf0, buf1, sem0, sem1):
    nc = plsc.get_sparse_core_info().num_cores
    b = lax.axis_index("s") * nc + lax.axis_index("c")
    bufs = (buf0, buf1)
    sems = (sem0, sem1)

    def chunk_acc(buf, accs):
        def row(r, a):
            return tuple(a[j] + buf[r, pl.ds(j * 16, 16)] for j in range(_NV))
        return lax.fori_loop(0, _RCHUNK, row, accs)

    accs = tuple(jnp.zeros((16,), jnp.float32) for _ in range(_NV))
    cps = [None, None]
    cps[0] = pltpu.async_copy(x_hbm.at[b, pl.ds(0, _RCHUNK)], buf0, sem0)
    for c in range(_NCHUNK):
        nxt = c + 1
        if nxt < _NCHUNK:
            cps[nxt % 2] = pltpu.async_copy(
                x_hbm.at[b, pl.ds(nxt * _RCHUNK, _RCHUNK)],
                bufs[nxt % 2], sems[nxt % 2])
        cps[c % 2].wait()
        accs = chunk_acc(bufs[c % 2], accs)
    for j in range(_NV):
        buf0[0, pl.ds(j * 16, 16)] = accs[j]
    pltpu.sync_copy(buf0.at[0], out_hbm.at[b])


_sc_pool = functools.partial(
    pl.kernel,
    mesh=plsc.VectorSubcoreMesh(core_axis_name="c", subcore_axis_name="s"),
    out_type=jax.ShapeDtypeStruct((_BSC, _C), jnp.float32),
    scratch_types=[
        pltpu.VMEM((_RCHUNK, _C), jnp.float32),
        pltpu.VMEM((_RCHUNK, _C), jnp.float32),
        pltpu.SemaphoreType.DMA,
        pltpu.SemaphoreType.DMA,
    ],
)(_sc_pool_body)


def _mlp(pooled, w1_ref, b1_ref, w2_ref, b2_ref):
    h = pooled @ w1_ref[...] + b1_ref[...]
    h = 0.5 * h * (1.0 + jax.lax.erf(h * (2.0 ** -0.5)))
    logits = h @ w2_ref[...] + b2_ref[...]
    m = jnp.max(logits, axis=-1, keepdims=True)
    e = jnp.exp(logits - m)
    return e / jnp.sum(e, axis=-1, keepdims=True)


def _tc_router_body(x_ref, w1_ref, b1_ref, w2_ref, b2_ref, out_ref):
    hw = x_ref.shape[1]
    pooled = jnp.sum(x_ref[...], axis=1) * (1.0 / hw)
    out_ref[0, :, :] = _mlp(pooled, w1_ref, b1_ref, w2_ref, b2_ref)


def _sc_mlp_body(sums_ref, w1_ref, b1_ref, w2_ref, b2_ref, out_ref):
    pooled = sums_ref[...] * (1.0 / _HW)
    out_ref[...] = _mlp(pooled, w1_ref, b1_ref, w2_ref, b2_ref)


def kernel(x, W1, b1, W2, b2):
    B, C, H, W = x.shape
    hw = H * W
    E = W2.shape[1]
    xt = jnp.transpose(x, (0, 2, 3, 1)).reshape(B, hw, C)

    sc_sums = _sc_pool(xt)

    n_tc = (B - _BSC) // _BBLK
    out_tc = pl.pallas_call(
        _tc_router_body,
        grid=(n_tc,),
        in_specs=[
            pl.BlockSpec((_BBLK, hw, C), lambda i: (i + _BSC // _BBLK, 0, 0)),
            pl.BlockSpec((C, W1.shape[1]), lambda i: (0, 0)),
            pl.BlockSpec((W1.shape[1],), lambda i: (0,)),
            pl.BlockSpec((W1.shape[1], E), lambda i: (0, 0)),
            pl.BlockSpec((E,), lambda i: (0,)),
        ],
        out_specs=pl.BlockSpec((1, _BBLK, E), lambda i: (i, 0, 0)),
        out_shape=jax.ShapeDtypeStruct((n_tc, _BBLK, E), jnp.float32),
    )(xt, W1, b1, W2, b2)

    out_sc = pl.pallas_call(
        _sc_mlp_body,
        out_shape=jax.ShapeDtypeStruct((_BSC, E), jnp.float32),
    )(sc_sums, W1, b1, W2, b2)

    return jnp.concatenate([out_sc, out_tc.reshape(B - _BSC, E)], axis=0)


# speedup vs baseline: 1.0247x; 1.0247x over previous
import functools

import jax
import jax.numpy as jnp
from jax import lax
from jax.experimental import pallas as pl
from jax.experimental.pallas import tpu as pltpu
from jax.experimental.pallas import tpu_sc as plsc

_B, _C, _HW = 64, 768, 576
_BSC = 16
_BBLK = 4
_RCHUNK = 48
_RHALF = _HW // 2
_NCHUNK = _RHALF // _RCHUNK
_NV = _C // 16


def _sc_pool_body(x_hbm, out_hbm, buf0, buf1, sem0, sem1):
    nc = plsc.get_sparse_core_info().num_cores
    w = lax.axis_index("s") * nc + lax.axis_index("c")
    b = w // 2
    r0 = (w % 2) * _RHALF
    bufs = (buf0, buf1)
    sems = (sem0, sem1)

    def chunk_acc(buf, accs):
        def row(r, a):
            return tuple(a[j] + buf[r, pl.ds(j * 16, 16)] for j in range(_NV))
        return lax.fori_loop(0, _RCHUNK, row, accs)

    accs = tuple(jnp.zeros((16,), jnp.float32) for _ in range(_NV))
    cps = [None, None]
    cps[0] = pltpu.async_copy(x_hbm.at[b, pl.ds(r0, _RCHUNK)], buf0, sem0)
    for c in range(_NCHUNK):
        nxt = c + 1
        if nxt < _NCHUNK:
            cps[nxt % 2] = pltpu.async_copy(
                x_hbm.at[b, pl.ds(r0 + nxt * _RCHUNK, _RCHUNK)],
                bufs[nxt % 2], sems[nxt % 2])
        cps[c % 2].wait()
        accs = chunk_acc(bufs[c % 2], accs)
    for j in range(_NV):
        buf0[0, pl.ds(j * 16, 16)] = accs[j]
    pltpu.sync_copy(buf0.at[0], out_hbm.at[w])


_sc_pool = functools.partial(
    pl.kernel,
    mesh=plsc.VectorSubcoreMesh(core_axis_name="c", subcore_axis_name="s"),
    out_type=jax.ShapeDtypeStruct((2 * _BSC, _C), jnp.float32),
    scratch_types=[
        pltpu.VMEM((_RCHUNK, _C), jnp.float32),
        pltpu.VMEM((_RCHUNK, _C), jnp.float32),
        pltpu.SemaphoreType.DMA,
        pltpu.SemaphoreType.DMA,
    ],
)(_sc_pool_body)


def _mlp(pooled, w1_ref, b1_ref, w2_ref, b2_ref):
    h = pooled @ w1_ref[...] + b1_ref[...]
    h = 0.5 * h * (1.0 + jax.lax.erf(h * (2.0 ** -0.5)))
    logits = h @ w2_ref[...] + b2_ref[...]
    m = jnp.max(logits, axis=-1, keepdims=True)
    e = jnp.exp(logits - m)
    return e / jnp.sum(e, axis=-1, keepdims=True)


def _tc_router_body(x_ref, w1_ref, b1_ref, w2_ref, b2_ref, out_ref):
    hw = x_ref.shape[1]
    pooled = jnp.sum(x_ref[...], axis=1) * (1.0 / hw)
    out_ref[0, :, :] = _mlp(pooled, w1_ref, b1_ref, w2_ref, b2_ref)


def _sc_mlp_body(sums_ref, w1_ref, b1_ref, w2_ref, b2_ref, out_ref):
    pooled = jnp.sum(sums_ref[...], axis=1) * (1.0 / _HW)
    out_ref[...] = _mlp(pooled, w1_ref, b1_ref, w2_ref, b2_ref)


def kernel(x, W1, b1, W2, b2):
    B, C, H, W = x.shape
    hw = H * W
    E = W2.shape[1]
    xt = jnp.transpose(x, (0, 2, 3, 1)).reshape(B, hw, C)

    sc_sums = _sc_pool(xt)

    n_tc = (B - _BSC) // _BBLK
    out_tc = pl.pallas_call(
        _tc_router_body,
        grid=(n_tc,),
        in_specs=[
            pl.BlockSpec((_BBLK, hw, C), lambda i: (i + _BSC // _BBLK, 0, 0)),
            pl.BlockSpec((C, W1.shape[1]), lambda i: (0, 0)),
            pl.BlockSpec((W1.shape[1],), lambda i: (0,)),
            pl.BlockSpec((W1.shape[1], E), lambda i: (0, 0)),
            pl.BlockSpec((E,), lambda i: (0,)),
        ],
        out_specs=pl.BlockSpec((1, _BBLK, E), lambda i: (i, 0, 0)),
        out_shape=jax.ShapeDtypeStruct((n_tc, _BBLK, E), jnp.float32),
    )(xt, W1, b1, W2, b2)

    out_sc = pl.pallas_call(
        _sc_mlp_body,
        out_shape=jax.ShapeDtypeStruct((_BSC, E), jnp.float32),
    )(sc_sums.reshape(_BSC, 2, C), W1, b1, W2, b2)

    return jnp.concatenate([out_sc, out_tc.reshape(B - _BSC, E)], axis=0)


# speedup vs baseline: 1.2043x; 1.1753x over previous
import jax
import jax.numpy as jnp
from jax.experimental import pallas as pl
from jax.experimental.pallas import tpu as pltpu

_BBLK = 2


def _router_body(x_ref, w1_ref, b1_ref, w2_ref, b2_ref, out_ref):
    hw = x_ref.shape[1]
    pooled = jnp.sum(x_ref[...], axis=1) * (1.0 / hw)
    h = pooled @ w1_ref[...] + b1_ref[...]
    h = 0.5 * h * (1.0 + jax.lax.erf(h * (2.0 ** -0.5)))
    logits = h @ w2_ref[...] + b2_ref[...]
    m = jnp.max(logits, axis=-1, keepdims=True)
    e = jnp.exp(logits - m)
    out_ref[0, :, :] = e / jnp.sum(e, axis=-1, keepdims=True)


def kernel(x, W1, b1, W2, b2):
    B, C, H, W = x.shape
    hw = H * W
    E = W2.shape[1]
    xt = jnp.transpose(x, (0, 2, 3, 1)).reshape(B, hw, C)
    grid = (B // _BBLK,)
    out = pl.pallas_call(
        _router_body,
        grid=grid,
        in_specs=[
            pl.BlockSpec((_BBLK, hw, C), lambda i: (i, 0, 0)),
            pl.BlockSpec((C, W1.shape[1]), lambda i: (0, 0)),
            pl.BlockSpec((W1.shape[1],), lambda i: (0,)),
            pl.BlockSpec((W1.shape[1], E), lambda i: (0, 0)),
            pl.BlockSpec((E,), lambda i: (0,)),
        ],
        out_specs=pl.BlockSpec((1, _BBLK, E), lambda i: (i, 0, 0)),
        out_shape=jax.ShapeDtypeStruct((B // _BBLK, _BBLK, E), jnp.float32),
    )(xt, W1, b1, W2, b2)
    return out.reshape(B, E)


# speedup vs baseline: 1.5390x; 1.2779x over previous
import jax
import jax.numpy as jnp
from jax.experimental import pallas as pl
from jax.experimental.pallas import tpu as pltpu

_BBLK = 4


def _router_body(x_ref, w1_ref, b1_ref, w2_ref, b2_ref, out_ref):
    hw = x_ref.shape[1]
    pooled = jnp.sum(x_ref[...], axis=1) * (1.0 / hw)
    h = pooled @ w1_ref[...] + b1_ref[...]
    h = 0.5 * h * (1.0 + jax.lax.erf(h * (2.0 ** -0.5)))
    logits = h @ w2_ref[...] + b2_ref[...]
    m = jnp.max(logits, axis=-1, keepdims=True)
    e = jnp.exp(logits - m)
    out_ref[0, :, :] = e / jnp.sum(e, axis=-1, keepdims=True)


def kernel(x, W1, b1, W2, b2):
    B, C, H, W = x.shape
    hw = H * W
    E = W2.shape[1]
    xt = jnp.transpose(x, (0, 2, 3, 1)).reshape(B, hw, C)
    grid = (B // _BBLK,)
    out = pl.pallas_call(
        _router_body,
        grid=grid,
        in_specs=[
            pl.BlockSpec((_BBLK, hw, C), lambda i: (i, 0, 0)),
            pl.BlockSpec((C, W1.shape[1]), lambda i: (0, 0)),
            pl.BlockSpec((W1.shape[1],), lambda i: (0,)),
            pl.BlockSpec((W1.shape[1], E), lambda i: (0, 0)),
            pl.BlockSpec((E,), lambda i: (0,)),
        ],
        out_specs=pl.BlockSpec((1, _BBLK, E), lambda i: (i, 0, 0)),
        out_shape=jax.ShapeDtypeStruct((B // _BBLK, _BBLK, E), jnp.float32),
    )(xt, W1, b1, W2, b2)
    return out.reshape(B, E)
